# Initial kernel scaffold; baseline (speedup 1.0000x reference)
import jax
import jax.numpy as jnp
from jax.experimental import pallas as pl


def kernel(input_ids, attention_mask, emb_table):
    raise NotImplementedError("write your pallas kernel here")


# speedup vs baseline: 2.0096x; 2.0096x over previous
import functools

import jax
import jax.numpy as jnp
from jax import lax
from jax.experimental import pallas as pl
from jax.experimental.pallas import tpu as pltpu
from jax.experimental.pallas import tpu_sc as plsc

_HID = 128
_NCORES = 2
_NSUB = 16
_NW = _NCORES * _NSUB
_GROW = 128
_CHUNK_GATHERS = 4
_C = _CHUNK_GATHERS * _GROW


def _sc_embed(ids2d, table):
    n_rows = ids2d.shape[0] * _GROW
    b_per_w = n_rows // _NW
    chunks = b_per_w // _C
    mesh = plsc.VectorSubcoreMesh(core_axis_name="c", subcore_axis_name="s")

    @functools.partial(
        pl.kernel,
        mesh=mesh,
        out_type=jax.ShapeDtypeStruct((n_rows, _HID), jnp.float32),
        scratch_types=[
            pltpu.VMEM((_CHUNK_GATHERS, _GROW), jnp.int32),
            pltpu.VMEM((_C, _HID), jnp.float32),
            pltpu.SemaphoreType.DMA,
        ],
    )
    def run(ids_hbm, table_hbm, out_hbm, idx_v, rows_v, sem):
        wid = lax.axis_index("s") * _NCORES + lax.axis_index("c")
        row0 = wid * b_per_w
        irow0 = wid * (b_per_w // _GROW)

        def body(i, carry):
            pltpu.sync_copy(
                ids_hbm.at[pl.ds(irow0 + i * _CHUNK_GATHERS, _CHUNK_GATHERS)],
                idx_v)
            copies = [
                pltpu.async_copy(
                    table_hbm.at[idx_v.at[j]],
                    rows_v.at[pl.ds(j * _GROW, _GROW)],
                    sem)
                for j in range(_CHUNK_GATHERS)
            ]
            for cp in copies:
                cp.wait()
            pltpu.sync_copy(rows_v, out_hbm.at[pl.ds(row0 + i * _C, _C)])
            return carry

        lax.fori_loop(0, chunks, body, 0)

    return run(ids2d, table)


def kernel(input_ids, attention_mask, emb_table):
    del attention_mask
    b, l = input_ids.shape
    n = b * l
    ids2d = input_ids.astype(jnp.int32).reshape(n // _GROW, _GROW)
    out = _sc_embed(ids2d, emb_table)
    return out.reshape(b, l, _HID)
